# baseline (speedup 1.0000x reference)
import functools
import math

import jax
import jax.numpy as jnp
from jax import lax
from jax.experimental import pallas as pl
from jax.experimental.pallas import tpu as pltpu
from jax.experimental.pallas import tpu_sc as plsc

_SMOOTH = 0.1
_PAD = 0

_R = 32768
_S = 8192
_V = 1000
_NW = 32
_RPW = _R // _NW
_CHUNK = 32
_NCH = _RPW // _CHUNK
_NFULL = _V // 16
_TAIL = _V - _NFULL * 16


def _sc_body(pred_hbm, tgt_hbm, out_hbm, tbuf, buf0, buf1, resbuf,
             sem0, sem1, *, eps, c0):
    nc = 2
    wid = lax.axis_index("s") * nc + lax.axis_index("c")
    wpb = _S // _RPW
    b0 = wid // wpb
    s0 = (wid % wpb) * _RPW

    pltpu.sync_copy(tgt_hbm.at[b0, pl.ds(s0, _RPW)], tbuf.at[pl.ds(0, _RPW)])

    pltpu.async_copy(pred_hbm.at[b0, pl.ds(s0, _CHUNK)], buf0, sem0)
    pltpu.async_copy(pred_hbm.at[b0, pl.ds(s0 + _CHUNK, _CHUNK)], buf1, sem1)

    iota16 = lax.iota(jnp.int32, 16)
    tailmask = jnp.where(iota16 < 8, 0.0, 1.0)
    zero16 = jnp.zeros((16,), jnp.float32)
    zeros_i = jnp.zeros((16,), jnp.int32)

    def process_chunk(jj, buf, accs):
        acc_s, acc_p0, acc_pt, acc_n1 = accs

        def row_body(r, acc_s):
            part = buf[r, pl.ds(0, 16)]
            for k in range(1, _NFULL):
                part = part + buf[r, pl.ds(16 * k, 16)]
            part = part + buf[r, pl.ds(_V - 16, 16)] * tailmask
            t = tbuf[pl.ds(jj * _CHUNK + r, 16)][0]
            return acc_s + jnp.where(t != _PAD, part, zero16)

        acc_s = lax.fori_loop(0, _CHUNK, row_body, acc_s, unroll=False)

        for g in range(_CHUNK // 16):
            rows16 = iota16 + g * 16
            t16 = tbuf[pl.ds(jj * _CHUNK + g * 16, 16)]
            ptv = plsc.load_gather(buf, [rows16, t16])
            p0v = plsc.load_gather(buf, [rows16, zeros_i])
            m = t16 != _PAD
            acc_pt = acc_pt + jnp.where(m, ptv, zero16)
            acc_p0 = acc_p0 + jnp.where(m, p0v, zero16)
            acc_n1 = acc_n1 + jnp.where(m, 1.0, 0.0)
        return acc_s, acc_p0, acc_pt, acc_n1

    def outer(j2, accs):
        for b in range(2):
            jj = 2 * j2 + b
            buf = buf0 if b == 0 else buf1
            sem = sem0 if b == 0 else sem1
            pltpu.make_async_copy(
                pred_hbm.at[b0, pl.ds(s0 + jj * _CHUNK, _CHUNK)], buf, sem
            ).wait()
            accs = process_chunk(jj, buf, accs)

            @pl.when(jj + 2 < _NCH)
            def _():
                pltpu.async_copy(
                    pred_hbm.at[b0, pl.ds(s0 + (jj + 2) * _CHUNK, _CHUNK)],
                    buf, sem)
        return accs

    init = (jnp.zeros((16,), jnp.float32), jnp.zeros((16,), jnp.float32),
            jnp.zeros((16,), jnp.float32), jnp.zeros((16,), jnp.float32))
    acc_s, acc_p0, acc_pt, acc_n1 = lax.fori_loop(
        0, _NCH // 2, outer, init, unroll=False)

    res = (c0 * acc_n1 - eps * (acc_s - acc_p0)
           + (eps - (1.0 - _SMOOTH)) * acc_pt)
    resbuf[...] = res
    pltpu.sync_copy(resbuf, out_hbm.at[wid])


def kernel(pred, target):
    B, S, V = pred.shape
    eps = _SMOOTH / (V - 2)
    c0 = _SMOOTH * math.log(eps) + (1.0 - _SMOOTH) * math.log(1.0 - _SMOOTH)

    mesh = plsc.VectorSubcoreMesh(core_axis_name="c", subcore_axis_name="s")
    sc_fn = functools.partial(
        pl.kernel,
        mesh=mesh,
        out_type=jax.ShapeDtypeStruct((_NW, 16), jnp.float32),
        scratch_types=[
            pltpu.VMEM((_RPW + 16,), jnp.int32),
            pltpu.VMEM((_CHUNK, _V), jnp.float32),
            pltpu.VMEM((_CHUNK, _V), jnp.float32),
            pltpu.VMEM((16,), jnp.float32),
            pltpu.SemaphoreType.DMA,
            pltpu.SemaphoreType.DMA,
        ],
        compiler_params=pltpu.CompilerParams(use_tc_tiling_on_sc=True, needs_layout_passes=False),
    )(functools.partial(_sc_body, eps=eps, c0=c0))
    partials = sc_fn(pred, target)
    return jnp.sum(partials)


# speedup vs baseline: 1.0034x; 1.0034x over previous
import functools
import math

import jax
import jax.numpy as jnp
from jax import lax
from jax.experimental import pallas as pl
from jax.experimental.pallas import tpu as pltpu
from jax.experimental.pallas import tpu_sc as plsc

_SMOOTH = 0.1
_PAD = 0

_B = 4
_S = 8192
_R = _B * _S
_V = 1000

_SC_ROWS = 12288
_TC_ROWS = _R - _SC_ROWS
_RB = 2048
_G = _TC_ROWS // _RB

_NW = 32
_RPW = _SC_ROWS // _NW
_CHUNK = 32
_NCH = _RPW // _CHUNK
_NFULL = _V // 16


def _tc_body(tgt_row_ref, tgt_col_ref, pred_ref, out_ref,
             wacc_ref, sacc_ref, nacc_ref, *, nsteps, eps, c0):
    i = pl.program_id(0)

    @pl.when(i == 0)
    def _():
        wacc_ref[...] = jnp.zeros_like(wacc_ref)
        sacc_ref[...] = jnp.zeros_like(sacc_ref)
        nacc_ref[...] = jnp.zeros_like(nacc_ref)

    pred = pred_ref[...]
    tgt_row = tgt_row_ref[0]
    tgt_col = tgt_col_ref[...]

    a = (tgt_row != _PAD).astype(jnp.float32)
    wacc_ref[...] += jax.lax.dot_general(
        a, pred, (((1,), (0,)), ((), ())),
        preferred_element_type=jnp.float32)
    nacc_ref[...] += jnp.sum(a, keepdims=True)

    t_adj = jnp.where(tgt_col == _PAD, -1, tgt_col)
    cols = jax.lax.broadcasted_iota(jnp.int32, pred.shape, 1)
    ptsel = jnp.where(cols == t_adj, pred, 0.0)
    sacc_ref[...] += jnp.sum(ptsel, keepdims=True)

    @pl.when(i == nsteps - 1)
    def _():
        wsum = jnp.sum(wacc_ref[...], keepdims=True)
        w0 = wacc_ref[:, 0:1]
        out_ref[...] = (c0 * nacc_ref[...]
                        - eps * (wsum - w0)
                        + (eps - (1.0 - _SMOOTH)) * sacc_ref[...])


def _tc_part(pred2, tgt_row, tgt_col, eps, c0):
    return pl.pallas_call(
        functools.partial(_tc_body, nsteps=_G, eps=eps, c0=c0),
        grid=(_G,),
        in_specs=[
            pl.BlockSpec((1, 1, _RB), lambda i: (i, 0, 0)),
            pl.BlockSpec((_RB, 1), lambda i: (i, 0)),
            pl.BlockSpec((_RB, _V), lambda i: (i, 0)),
        ],
        out_specs=pl.BlockSpec((1, 1), lambda i: (0, 0)),
        out_shape=jax.ShapeDtypeStruct((1, 1), jnp.float32),
        scratch_shapes=[
            pltpu.VMEM((1, _V), jnp.float32),
            pltpu.VMEM((1, 1), jnp.float32),
            pltpu.VMEM((1, 1), jnp.float32),
        ],
    )(tgt_row, tgt_col, pred2)


def _sc_body(pred_hbm, tgt_hbm, out_hbm, tbuf, buf0, buf1, resbuf,
             sem0, sem1, *, eps, c0):
    nc = 2
    wid = lax.axis_index("s") * nc + lax.axis_index("c")
    base = _TC_ROWS + wid * _RPW

    def chunk_src(jj):
        row = base + jj * _CHUNK
        b0 = lax.shift_right_logical(row, 13)
        s0 = lax.bitwise_and(row, _S - 1)
        s0 = pl.multiple_of(s0, _CHUNK)
        return pred_hbm.at[b0, pl.ds(s0, _CHUNK)]

    tb = pl.multiple_of(base, _CHUNK)
    pltpu.sync_copy(tgt_hbm.at[pl.ds(tb, _RPW)], tbuf.at[pl.ds(0, _RPW)])

    pltpu.async_copy(chunk_src(0), buf0, sem0)
    pltpu.async_copy(chunk_src(1), buf1, sem1)

    iota16 = lax.iota(jnp.int32, 16)
    tailmask = jnp.where(iota16 < 8, 0.0, 1.0)
    zero16 = jnp.zeros((16,), jnp.float32)
    zeros_i = jnp.zeros((16,), jnp.int32)

    def process_chunk(jj, buf, accs):
        acc_s, acc_p0, acc_pt, acc_n1 = accs

        def row_body(r, acc_s):
            part = buf[r, pl.ds(0, 16)]
            for k in range(1, _NFULL):
                part = part + buf[r, pl.ds(16 * k, 16)]
            part = part + buf[r, pl.ds(_V - 16, 16)] * tailmask
            t = tbuf[pl.ds(jj * _CHUNK + r, 16)][0]
            return acc_s + jnp.where(t != _PAD, part, zero16)

        acc_s = lax.fori_loop(0, _CHUNK, row_body, acc_s, unroll=False)

        for g in range(_CHUNK // 16):
            rows16 = iota16 + g * 16
            t16 = tbuf[pl.ds(jj * _CHUNK + g * 16, 16)]
            ptv = plsc.load_gather(buf, [rows16, t16])
            p0v = plsc.load_gather(buf, [rows16, zeros_i])
            m = t16 != _PAD
            acc_pt = acc_pt + jnp.where(m, ptv, zero16)
            acc_p0 = acc_p0 + jnp.where(m, p0v, zero16)
            acc_n1 = acc_n1 + jnp.where(m, 1.0, 0.0)
        return acc_s, acc_p0, acc_pt, acc_n1

    def outer(j2, accs):
        for b in range(2):
            jj = 2 * j2 + b
            buf = buf0 if b == 0 else buf1
            sem = sem0 if b == 0 else sem1
            pltpu.make_async_copy(chunk_src(jj), buf, sem).wait()
            accs = process_chunk(jj, buf, accs)

            @pl.when(jj + 2 < _NCH)
            def _():
                pltpu.async_copy(chunk_src(jj + 2), buf, sem)
        return accs

    init = (jnp.zeros((16,), jnp.float32), jnp.zeros((16,), jnp.float32),
            jnp.zeros((16,), jnp.float32), jnp.zeros((16,), jnp.float32))
    acc_s, acc_p0, acc_pt, acc_n1 = lax.fori_loop(
        0, _NCH // 2, outer, init, unroll=False)

    res = (c0 * acc_n1 - eps * (acc_s - acc_p0)
           + (eps - (1.0 - _SMOOTH)) * acc_pt)
    resbuf[...] = res
    pltpu.sync_copy(resbuf, out_hbm.at[wid])


def _sc_part(pred, tgt_flat, eps, c0):
    mesh = plsc.VectorSubcoreMesh(core_axis_name="c", subcore_axis_name="s")
    sc_fn = functools.partial(
        pl.kernel,
        mesh=mesh,
        out_type=jax.ShapeDtypeStruct((_NW, 16), jnp.float32),
        scratch_types=[
            pltpu.VMEM((_RPW + 16,), jnp.int32),
            pltpu.VMEM((_CHUNK, _V), jnp.float32),
            pltpu.VMEM((_CHUNK, _V), jnp.float32),
            pltpu.VMEM((16,), jnp.float32),
            pltpu.SemaphoreType.DMA,
            pltpu.SemaphoreType.DMA,
        ],
        compiler_params=pltpu.CompilerParams(
            use_tc_tiling_on_sc=True, needs_layout_passes=False),
    )(functools.partial(_sc_body, eps=eps, c0=c0))
    return sc_fn(pred, tgt_flat)


def kernel(pred, target):
    B, S, V = pred.shape
    eps = _SMOOTH / (V - 2)
    c0 = _SMOOTH * math.log(eps) + (1.0 - _SMOOTH) * math.log(1.0 - _SMOOTH)

    pred2 = pred.reshape(B * S, V)
    tgt_row = target.reshape(B * S // _RB, 1, _RB)
    tgt_col = target.reshape(B * S, 1)
    tgt_flat = target.reshape(B * S)

    sc_partials = _sc_part(pred, tgt_flat, eps, c0)
    tc_out = _tc_part(pred2, tgt_row, tgt_col, eps, c0)
    return tc_out[0, 0] + jnp.sum(sc_partials)
